# Initial kernel scaffold; baseline (speedup 1.0000x reference)
import jax
import jax.numpy as jnp
from jax.experimental import pallas as pl


def kernel(inp, index):
    raise NotImplementedError("write your pallas kernel here")


# speedup vs baseline: 22.2738x; 22.2738x over previous
import jax
import jax.numpy as jnp
from jax.experimental import pallas as pl
from jax.experimental.pallas import tpu as pltpu

_LANES = 128


def _gather_body(x_ref, i_ref, o_ref):
    x = x_ref[...]
    idx = i_ref[...]
    n_chunks = x.shape[1] // _LANES
    n_p = idx.shape[1] // _LANES

    lo = jnp.bitwise_and(idx, _LANES - 1)
    hi = jnp.right_shift(idx, 7)

    for h in range(n_p):
        sl = slice(h * _LANES, (h + 1) * _LANES)
        lo_h = lo[:, sl]
        hi_h = hi[:, sl]
        acc = jnp.take_along_axis(x[:, 0:_LANES], lo_h, axis=1)
        for c in range(1, n_chunks):
            g = jnp.take_along_axis(x[:, c * _LANES:(c + 1) * _LANES],
                                    lo_h, axis=1)
            acc = jnp.where(hi_h == c, g, acc)
        o_ref[:, sl] = acc


def _gather_2d(x2d, idx2d, tile_r=256):
    R, M = x2d.shape
    _, P = idx2d.shape
    assert M % _LANES == 0 and P % _LANES == 0 and R % tile_r == 0

    grid = (R // tile_r,)
    return pl.pallas_call(
        _gather_body,
        out_shape=jax.ShapeDtypeStruct((R, P), x2d.dtype),
        grid=grid,
        in_specs=[
            pl.BlockSpec((tile_r, M), lambda i: (i, 0)),
            pl.BlockSpec((tile_r, P), lambda i: (i, 0)),
        ],
        out_specs=pl.BlockSpec((tile_r, P), lambda i: (i, 0)),
        compiler_params=pltpu.CompilerParams(
            dimension_semantics=("parallel",),
        ),
    )(x2d, idx2d)


def kernel(inp, index):
    batch_shape = index.shape[:-1]
    M = inp.shape[-1]
    P = index.shape[-1]
    x2 = inp.reshape(-1, M)
    i2 = index.reshape(-1, P).astype(jnp.int32)
    out2 = _gather_2d(x2, i2)
    return out2.reshape(*batch_shape, P).astype(inp.dtype)


# speedup vs baseline: 28.6602x; 1.2867x over previous
import jax
import jax.numpy as jnp
from jax.experimental import pallas as pl
from jax.experimental.pallas import tpu as pltpu

_LANES = 128


def _gather_body(x_ref, i_ref, o_ref):
    x = x_ref[...]
    idx = i_ref[...]
    n_chunks = x.shape[1] // _LANES
    n_p = idx.shape[1] // _LANES

    lo = jnp.bitwise_and(idx, _LANES - 1)
    hi = jnp.right_shift(idx, 7)

    for h in range(n_p):
        sl = slice(h * _LANES, (h + 1) * _LANES)
        lo_h = lo[:, sl]
        hi_h = hi[:, sl]
        acc = jnp.take_along_axis(x[:, 0:_LANES], lo_h, axis=1)
        for c in range(1, n_chunks):
            g = jnp.take_along_axis(x[:, c * _LANES:(c + 1) * _LANES],
                                    lo_h, axis=1)
            acc = jnp.where(hi_h == c, g, acc)
        o_ref[:, sl] = acc


def _gather_2d(x2d, idx2d, tile_r=512):
    R, M = x2d.shape
    _, P = idx2d.shape
    assert M % _LANES == 0 and P % _LANES == 0 and R % tile_r == 0

    grid = (R // tile_r,)
    return pl.pallas_call(
        _gather_body,
        out_shape=jax.ShapeDtypeStruct((R, P), x2d.dtype),
        grid=grid,
        in_specs=[
            pl.BlockSpec((tile_r, M), lambda i: (i, 0)),
            pl.BlockSpec((tile_r, P), lambda i: (i, 0)),
        ],
        out_specs=pl.BlockSpec((tile_r, P), lambda i: (i, 0)),
        compiler_params=pltpu.CompilerParams(
            dimension_semantics=("parallel",),
        ),
    )(x2d, idx2d)


def kernel(inp, index):
    batch_shape = index.shape[:-1]
    M = inp.shape[-1]
    P = index.shape[-1]
    x2 = inp.reshape(-1, M)
    i2 = index.reshape(-1, P).astype(jnp.int32)
    out2 = _gather_2d(x2, i2)
    return out2.reshape(*batch_shape, P).astype(inp.dtype)


# speedup vs baseline: 30.7747x; 1.0738x over previous
import jax
import jax.numpy as jnp
from jax.experimental import pallas as pl
from jax.experimental.pallas import tpu as pltpu

_LANES = 128


def _gather_body(x_ref, i_ref, o_ref):
    x = x_ref[...]
    idx = i_ref[...]
    n_chunks = x.shape[1] // _LANES
    n_p = idx.shape[1] // _LANES

    lo = jnp.bitwise_and(idx, _LANES - 1)
    hi = jnp.right_shift(idx, 7)

    for h in range(n_p):
        sl = slice(h * _LANES, (h + 1) * _LANES)
        lo_h = lo[:, sl]
        hi_h = hi[:, sl]
        acc = jnp.take_along_axis(x[:, 0:_LANES], lo_h, axis=1)
        for c in range(1, n_chunks):
            g = jnp.take_along_axis(x[:, c * _LANES:(c + 1) * _LANES],
                                    lo_h, axis=1)
            acc = jnp.where(hi_h == c, g, acc)
        o_ref[:, sl] = acc


def _gather_2d(x2d, idx2d, tile_r=1024):
    R, M = x2d.shape
    _, P = idx2d.shape
    assert M % _LANES == 0 and P % _LANES == 0 and R % tile_r == 0

    grid = (R // tile_r,)
    return pl.pallas_call(
        _gather_body,
        out_shape=jax.ShapeDtypeStruct((R, P), x2d.dtype),
        grid=grid,
        in_specs=[
            pl.BlockSpec((tile_r, M), lambda i: (i, 0)),
            pl.BlockSpec((tile_r, P), lambda i: (i, 0)),
        ],
        out_specs=pl.BlockSpec((tile_r, P), lambda i: (i, 0)),
        compiler_params=pltpu.CompilerParams(
            dimension_semantics=("parallel",),
        ),
    )(x2d, idx2d)


def kernel(inp, index):
    batch_shape = index.shape[:-1]
    M = inp.shape[-1]
    P = index.shape[-1]
    x2 = inp.reshape(-1, M)
    i2 = index.reshape(-1, P).astype(jnp.int32)
    out2 = _gather_2d(x2, i2)
    return out2.reshape(*batch_shape, P).astype(inp.dtype)
